# baseline (device time: 32789 ns/iter reference)
import jax
import jax.numpy as jnp
from jax import lax
from jax.experimental import pallas as pl
from jax.experimental.pallas import tpu as pltpu

K = 16
NDEV = 8
KEY_MIN = jnp.iinfo(jnp.int32).min

OFFSETS = [
    (dx, dy, dz)
    for dx in (0, 1)
    for dy in (0, 1)
    for dz in (0, 1)
    if (dx, dy, dz) != (0, 0, 0)
]


def _encode(v, iota, n, idx_bits):
    i = lax.bitcast_convert_type(v, jnp.int32)
    key = jnp.where(i < 0, i ^ jnp.int32(0x7FFFFFFF), i)
    return (key & jnp.int32(~((1 << idx_bits) - 1))) | ((n - 1) - iota)


def _decode(key, idx_bits):
    k0 = key & jnp.int32(~((1 << idx_bits) - 1))
    i = jnp.where(k0 < 0, k0 ^ jnp.int32(0x7FFFFFFF), k0)
    return lax.bitcast_convert_type(i, jnp.float32)


def _extract16(values, n, idx_bits, iota_k):
    m = values.shape[0]
    iota = lax.broadcasted_iota(jnp.int32, (m, n), 1)
    keys = _encode(values, iota, n, idx_bits)
    acc = jnp.full((m, K), KEY_MIN, jnp.int32)
    for i in range(K):
        mk = jnp.max(keys, axis=1, keepdims=True)
        acc = jnp.where(iota_k == i, mk, acc)
        keys = jnp.where(keys == mk, KEY_MIN, keys)
    return _decode(acc, idx_bits)


def kernel(x):
    m, n = x.shape
    nq = n // 4
    qbits = (nq - 1).bit_length()

    def body(x_ref, out_ref, xq_ref, sbuf, rbuf, copy_sem, send_sems, recv_sems):
        my_x = lax.axis_index("x")
        my_y = lax.axis_index("y")
        my_z = lax.axis_index("z")

        def peer(off):
            dx, dy, dz = off
            return (
                (1 - my_x) if dx else my_x,
                (1 - my_y) if dy else my_y,
                (1 - my_z) if dz else my_z,
            )

        q = 2 * my_x + my_z
        copy = pltpu.make_async_copy(x_ref.at[:, pl.ds(q * nq, nq)], xq_ref, copy_sem)
        copy.start()

        barrier_sem = pltpu.get_barrier_semaphore()
        for off in OFFSETS:
            pl.semaphore_signal(
                barrier_sem, inc=1, device_id=peer(off),
                device_id_type=pl.DeviceIdType.MESH,
            )
        copy.wait()

        iota_k = lax.broadcasted_iota(jnp.int32, (m, K), 1)
        sbuf[:, :] = _extract16(xq_ref[:, :], nq, qbits, iota_k)

        pl.semaphore_wait(barrier_sem, len(OFFSETS))

        rdmas = []
        for s, off in enumerate(OFFSETS):
            r = pltpu.make_async_remote_copy(
                src_ref=sbuf,
                dst_ref=rbuf.at[s],
                send_sem=send_sems.at[s],
                recv_sem=recv_sems.at[s],
                device_id=peer(off),
                device_id_type=pl.DeviceIdType.MESH,
            )
            r.start()
            rdmas.append(r)
        for r in rdmas:
            r.wait_recv()

        cand = jnp.concatenate([sbuf[:, :]] + [rbuf[s] for s in range(7)], axis=1)
        out_ref[:, :] = _extract16(cand, NDEV * K, 7, iota_k)

        for r in rdmas:
            r.wait_send()

    return pl.pallas_call(
        body,
        out_shape=jax.ShapeDtypeStruct((m, K), jnp.float32),
        in_specs=[pl.BlockSpec(memory_space=pl.MemorySpace.ANY)],
        out_specs=pl.BlockSpec(memory_space=pltpu.VMEM),
        scratch_shapes=[
            pltpu.VMEM((m, nq), jnp.float32),
            pltpu.VMEM((m, K), jnp.float32),
            pltpu.VMEM((7, m, K), jnp.float32),
            pltpu.SemaphoreType.DMA,
            pltpu.SemaphoreType.DMA((7,)),
            pltpu.SemaphoreType.DMA((7,)),
        ],
        compiler_params=pltpu.CompilerParams(collective_id=0),
    )(x)


# device time: 28558 ns/iter; 1.1482x vs baseline; 1.1482x over previous
import jax
import jax.numpy as jnp
from jax import lax
from jax.experimental import pallas as pl
from jax.experimental.pallas import tpu as pltpu

K = 16
KEY_MIN = jnp.iinfo(jnp.int32).min


def _encode(v, iota, n, idx_bits):
    i = lax.bitcast_convert_type(v, jnp.int32)
    key = jnp.where(i < 0, i ^ jnp.int32(0x7FFFFFFF), i)
    return (key & jnp.int32(~((1 << idx_bits) - 1))) | ((n - 1) - iota)


def _decode(key, idx_bits):
    k0 = key & jnp.int32(~((1 << idx_bits) - 1))
    i = jnp.where(k0 < 0, k0 ^ jnp.int32(0x7FFFFFFF), k0)
    return lax.bitcast_convert_type(i, jnp.float32)


def _extract16(values, n, idx_bits, iota_k, ascending=False):
    m = values.shape[0]
    iota = lax.broadcasted_iota(jnp.int32, (m, n), 1)
    keys = _encode(values, iota, n, idx_bits)
    acc = jnp.full((m, K), KEY_MIN, jnp.int32)
    for i in range(K):
        mk = jnp.max(keys, axis=1, keepdims=True)
        acc = jnp.where(iota_k == (K - 1 - i if ascending else i), mk, acc)
        keys = jnp.where(keys == mk, KEY_MIN, keys)
    return _decode(acc, idx_bits)


def _bitonic_sort16(s, iota_k, descending):
    for d in (8, 4, 2, 1):
        low = (iota_k & d) == 0
        p = jnp.where(low, pltpu.roll(s, K - d, 1), pltpu.roll(s, d, 1))
        big, small = jnp.maximum(s, p), jnp.minimum(s, p)
        s = jnp.where(low, big, small) if descending else jnp.where(low, small, big)
    return s


def kernel(x):
    m, n = x.shape
    nq = n // 4
    qbits = (nq - 1).bit_length()

    def body(x_ref, out_ref, xq_ref, sbuf, rbuf1, rbuf2, rbuf3,
             copy_sem, send_sems, recv_sems):
        my_x = lax.axis_index("x")
        my_y = lax.axis_index("y")
        my_z = lax.axis_index("z")
        x_nbr = (1 - my_x, my_y, my_z)
        y_nbr = (my_x, 1 - my_y, my_z)
        z_nbr = (my_x, my_y, 1 - my_z)

        q = 2 * my_x + my_z
        copy = pltpu.make_async_copy(x_ref.at[:, pl.ds(q * nq, nq)], xq_ref, copy_sem)
        copy.start()

        barrier_sem = pltpu.get_barrier_semaphore()
        for nbr in (x_nbr, y_nbr, z_nbr):
            pl.semaphore_signal(
                barrier_sem, inc=1, device_id=nbr, device_id_type=pl.DeviceIdType.MESH
            )
        copy.wait()

        iota_k = lax.broadcasted_iota(jnp.int32, (m, K), 1)
        sbuf[:, :] = _extract16(xq_ref[:, :], nq, qbits, iota_k, ascending=True)

        pl.semaphore_wait(barrier_sem, 3)

        def exchange(idx, nbr, src, dst):
            r = pltpu.make_async_remote_copy(
                src_ref=src, dst_ref=dst,
                send_sem=send_sems.at[idx], recv_sem=recv_sems.at[idx],
                device_id=nbr, device_id_type=pl.DeviceIdType.MESH,
            )
            r.start()
            return r

        r1x = exchange(0, x_nbr, sbuf, rbuf1.at[0])
        r1y = exchange(1, y_nbr, sbuf, rbuf1.at[1])
        r1z = exchange(2, z_nbr, sbuf, rbuf1.at[2])

        r1x.wait_recv()
        r2y = exchange(4, y_nbr, rbuf1.at[0], rbuf2.at[1])
        r1y.wait_recv()
        r2z = exchange(5, z_nbr, rbuf1.at[1], rbuf2.at[2])
        r1z.wait_recv()
        r2x = exchange(3, x_nbr, rbuf1.at[2], rbuf2.at[0])

        r2z.wait_recv()
        r3 = exchange(6, x_nbr, rbuf2.at[2], rbuf3)

        r2x.wait_recv()
        r2y.wait_recv()
        cand = jnp.concatenate(
            [sbuf[:, :], rbuf1[0], rbuf1[1], rbuf1[2], rbuf2[0], rbuf2[1], rbuf2[2]],
            axis=1,
        )
        a_desc = _extract16(cand, 7 * K, 7, iota_k)

        r3.wait_recv()
        top = jnp.maximum(a_desc, rbuf3[:, :])
        out_ref[:, :] = _bitonic_sort16(top, iota_k, descending=True)

        for r in (r1x, r1y, r1z, r2x, r2y, r2z, r3):
            r.wait_send()

    return pl.pallas_call(
        body,
        out_shape=jax.ShapeDtypeStruct((m, K), jnp.float32),
        in_specs=[pl.BlockSpec(memory_space=pl.MemorySpace.ANY)],
        out_specs=pl.BlockSpec(memory_space=pltpu.VMEM),
        scratch_shapes=[
            pltpu.VMEM((m, nq), jnp.float32),
            pltpu.VMEM((m, K), jnp.float32),
            pltpu.VMEM((3, m, K), jnp.float32),
            pltpu.VMEM((3, m, K), jnp.float32),
            pltpu.VMEM((m, K), jnp.float32),
            pltpu.SemaphoreType.DMA,
            pltpu.SemaphoreType.DMA((7,)),
            pltpu.SemaphoreType.DMA((7,)),
        ],
        compiler_params=pltpu.CompilerParams(collective_id=0),
    )(x)
